# Initial kernel scaffold; baseline (speedup 1.0000x reference)
import jax
import jax.numpy as jnp
from jax.experimental import pallas as pl


def kernel(x):
    raise NotImplementedError("write your pallas kernel here")


# speedup vs baseline: 17.6322x; 17.6322x over previous
import jax
import jax.numpy as jnp
from jax.experimental import pallas as pl

_K1 = 513
_N = 32768
_ROWS = 128
_BR = 16


def _body(x_ref, o_ref):
    x = x_ref[...]
    y = jnp.maximum(x, 0.0)
    bits = jax.lax.bitcast_convert_type(y, jnp.int32)

    def step(i, th):
        cand = th | (1 << (30 - i))
        cnt = jnp.sum((bits >= cand).astype(jnp.int32), axis=1, keepdims=True)
        return jnp.where(cnt >= _K1, cand, th)

    th = jax.lax.fori_loop(0, 31, step, jnp.zeros((_BR, 1), jnp.int32))
    thr = jax.lax.bitcast_convert_type(th, jnp.float32)
    o_ref[...] = jnp.maximum(x - thr, 0.0)


def kernel(x):
    return pl.pallas_call(
        _body,
        grid=(_ROWS // _BR,),
        in_specs=[pl.BlockSpec((_BR, _N), lambda i: (i, 0))],
        out_specs=pl.BlockSpec((_BR, _N), lambda i: (i, 0)),
        out_shape=jax.ShapeDtypeStruct((_ROWS, _N), jnp.float32),
    )(x)
